# baseline (speedup 1.0000x reference)
import jax
import jax.numpy as jnp
from jax import lax
from jax.experimental import pallas as pl
from jax.experimental.pallas import tpu as pltpu
from jax.experimental.pallas import tpu_sc as plsc

N = 10000
E = 320000
D = 128
DE = 16
T = 2

NC = 2
NS = 16
NW = NC * NS
K = 128
CH = 79
EP = NW * CH * K
NP = 10240
RPS = NP // NS
NZ = RPS // K
NEG = -1.0e30

_MESH = dict(core_axis_name="c", subcore_axis_name="s", num_cores=NC,
             num_subcores=NS)


def _deg_body(dst_hbm, out_hbm, idx_v, deg_v):
    c = lax.axis_index("c")
    s = lax.axis_index("s")
    w = c * NS + s

    def fill_zero(i, _):
        deg_v[pl.ds(i * 16, 16)] = jnp.zeros((16,), jnp.float32)
        return 0
    lax.fori_loop(0, NP // 16, fill_zero, 0)

    pltpu.sync_copy(dst_hbm.at[w], idx_v)

    ones = jnp.ones((16,), jnp.float32)

    def chunk(j, _):
        for g in range(K // 16):
            idx = idx_v[j, pl.ds(g * 16, 16)]
            plsc.addupdate_scatter(deg_v, [idx], ones)
        return 0
    lax.fori_loop(0, CH, chunk, 0)

    pltpu.sync_copy(deg_v, out_hbm.at[c, s])


_sc_deg = pl.kernel(
    _deg_body,
    out_type=jax.ShapeDtypeStruct((NC, NS, NP), jnp.float32),
    mesh=plsc.VectorSubcoreMesh(**_MESH),
    compiler_params=pltpu.CompilerParams(needs_layout_passes=False),
    scratch_types=[
        pltpu.VMEM((CH, K), jnp.int32),
        pltpu.VMEM((NP,), jnp.float32),
    ],
)


RNG = NP // NS
ACR = RNG + 16
TMP = 176


def _conv_body(hs_hbm, src_hbm, dst_hbm, out_hbm, src_v, dst_v, buf_v,
               acc_v, tsrc_v, tdst_v, sem):
    c = lax.axis_index("c")
    s = lax.axis_index("s")
    lo = s * RNG

    def fill_zero(i, _):
        acc_v[pl.ds(i * 16, 16)] = jnp.zeros((16,), jnp.float32)
        return 0
    lax.fori_loop(0, ACR * D // 16, fill_zero, 0)

    lane = lax.iota(jnp.int32, 16)
    dummy_src = jnp.zeros((16,), jnp.int32)
    dummy_dst = jnp.full((16,), RNG * D, jnp.int32)

    def drain(b, _):
        pltpu.async_copy(
            hs_hbm.at[tsrc_v.at[pl.ds(b * 16, 16)]], buf_v, sem).wait()
        base = tdst_v[pl.ds(b * 16, 16)]
        for col in range(D):
            v = plsc.load_gather(buf_v, [lane, jnp.full((16,), col, jnp.int32)])
            plsc.addupdate_scatter(acc_v, [base + col], v)
        return 0

    def worker(wi, ptr):
        w2 = c * NS + wi
        pltpu.sync_copy(src_hbm.at[w2], src_v)
        pltpu.sync_copy(dst_hbm.at[w2], dst_v)

        def chunk(j, ptr):
            for g in range(K // 16):
                srcv = src_v[j, pl.ds(g * 16, 16)]
                dstv = dst_v[j, pl.ds(g * 16, 16)]
                m = (dstv >= lo) & (dstv < lo + RNG)
                mi = m.astype(jnp.int32)
                pos = plsc.cumsum(mi) - mi + ptr
                plsc.store_scatter(tsrc_v, [pos], srcv, mask=m)
                plsc.store_scatter(tdst_v, [pos], (dstv - lo) * D, mask=m)
                ptr = ptr + jnp.sum(mi)
            nb = ptr // 16
            lax.fori_loop(0, nb, drain, 0)
            rem = ptr - nb * 16
            carry_src = tsrc_v[pl.ds(nb * 16, 16)]
            carry_dst = tdst_v[pl.ds(nb * 16, 16)]
            tsrc_v[pl.ds(0, 16)] = carry_src
            tdst_v[pl.ds(0, 16)] = carry_dst
            return rem

        return lax.fori_loop(0, CH, chunk, ptr)

    ptr = lax.fori_loop(0, NS, worker, jnp.int32(0))
    tsrc_v[pl.ds(ptr, 16)] = dummy_src
    tdst_v[pl.ds(ptr, 16)] = dummy_dst
    nb2 = (ptr + 15) // 16
    lax.fori_loop(0, nb2, drain, 0)
    pltpu.sync_copy(acc_v, out_hbm.at[c, s])


_sc_conv = pl.kernel(
    _conv_body,
    out_type=jax.ShapeDtypeStruct((NC, NS, ACR * D), jnp.float32),
    mesh=plsc.VectorSubcoreMesh(**_MESH),
    compiler_params=pltpu.CompilerParams(needs_layout_passes=False),
    scratch_types=[
        pltpu.VMEM((CH, K), jnp.int32),
        pltpu.VMEM((CH, K), jnp.int32),
        pltpu.VMEM((16, D), jnp.float32),
        pltpu.VMEM((ACR * D,), jnp.float32),
        pltpu.VMEM((TMP,), jnp.int32),
        pltpu.VMEM((TMP,), jnp.int32),
        pltpu.SemaphoreType.DMA,
    ],
)


def _edge_body(a_hbm, b_hbm, c_hbm, src_hbm, dst_hbm, out_hbm, src_v, dst_v,
               buf_a, buf_b, buf_c, out_v, sem_a, sem_b, sem_c):
    cc = lax.axis_index("c")
    s = lax.axis_index("s")
    w = cc * NS + s

    pltpu.sync_copy(src_hbm.at[w], src_v)
    pltpu.sync_copy(dst_hbm.at[w], dst_v)

    for g in range(8):
        out_v[0, pl.ds(g * 16, 16)] = jnp.zeros((16,), jnp.float32)

    def chunk(j, _):
        da = pltpu.async_copy(a_hbm.at[src_v.at[j]], buf_a, sem_a)
        db = pltpu.async_copy(b_hbm.at[dst_v.at[j]], buf_b, sem_b)
        dc = pltpu.async_copy(c_hbm.at[w, j], buf_c, sem_c)
        da.wait()
        db.wait()
        dc.wait()

        def edge(e, _):
            for g in range(8):
                av = buf_a[e, pl.ds(g * 16, 16)]
                bv = buf_b[e, pl.ds(g * 16, 16)]
                cv = buf_c[e, pl.ds(g * 16, 16)]
                out_v[0, pl.ds(g * 16, 16)] += jnp.maximum(av + bv + cv, 0.0)
            return 0
        return lax.fori_loop(0, K, edge, 0)

    lax.fori_loop(0, CH, chunk, 0)
    pltpu.sync_copy(out_v, out_hbm.at[w])


_sc_edge = pl.kernel(
    _edge_body,
    out_type=jax.ShapeDtypeStruct((NW, 1, D), jnp.float32),
    mesh=plsc.VectorSubcoreMesh(**_MESH),
    scratch_types=[
        pltpu.VMEM((CH, K), jnp.int32),
        pltpu.VMEM((CH, K), jnp.int32),
        pltpu.VMEM((K, D), jnp.float32),
        pltpu.VMEM((K, D), jnp.float32),
        pltpu.VMEM((K, D), jnp.float32),
        pltpu.VMEM((1, D), jnp.float32),
        pltpu.SemaphoreType.DMA,
        pltpu.SemaphoreType.DMA,
        pltpu.SemaphoreType.DMA,
    ],
)


RB = 1000
EB = 2048


def _prep_body(dp_ref, x_ref, w_ref, dinv_ref, hs_ref):
    deg = jnp.sum(dp_ref[...], axis=1, keepdims=True) + 1.0
    dinv = lax.rsqrt(deg)
    dinv_ref[...] = dinv
    h = jnp.dot(x_ref[...], w_ref[...], preferred_element_type=jnp.float32)
    hs_ref[...] = h * dinv


_tc_prep = pl.pallas_call(
    _prep_body,
    grid=(N // RB,),
    in_specs=[
        pl.BlockSpec((RB, NW), lambda i: (i, 0)),
        pl.BlockSpec((RB, D), lambda i: (i, 0)),
        pl.BlockSpec((D, D), lambda i: (0, 0)),
    ],
    out_specs=[
        pl.BlockSpec((RB, 1), lambda i: (i, 0)),
        pl.BlockSpec((RB, D), lambda i: (i, 0)),
    ],
    out_shape=[
        jax.ShapeDtypeStruct((N, 1), jnp.float32),
        jax.ShapeDtypeStruct((N, D), jnp.float32),
    ],
)


def _combine_body(agg_ref, hs_ref, dinv_ref, b_ref, wn_ref, out_ref):
    dinv = dinv_ref[...]
    xn = dinv * (agg_ref[0] + agg_ref[1] + hs_ref[...]) + b_ref[...]
    xn = jnp.maximum(xn, 0.0)
    out_ref[...] = jnp.dot(
        xn, wn_ref[...], preferred_element_type=jnp.float32) * dinv


_tc_combine = pl.pallas_call(
    _combine_body,
    grid=(N // RB,),
    in_specs=[
        pl.BlockSpec((NC, RB, D), lambda i: (0, i, 0)),
        pl.BlockSpec((RB, D), lambda i: (i, 0)),
        pl.BlockSpec((RB, 1), lambda i: (i, 0)),
        pl.BlockSpec((1, D), lambda i: (0, 0)),
        pl.BlockSpec((D, D), lambda i: (0, 0)),
    ],
    out_specs=pl.BlockSpec((RB, D), lambda i: (i, 0)),
    out_shape=jax.ShapeDtypeStruct((N, D), jnp.float32),
)


def _combine3_body(agg_ref, hs_ref, dinv_ref, b_ref, wa_ref, wb_ref, w1_ref,
                   a_ref, bt_ref, hsn_ref):
    dinv = dinv_ref[...]
    xn = dinv * (agg_ref[0] + agg_ref[1] + hs_ref[...]) + b_ref[...]
    xn = jnp.maximum(xn, 0.0)
    a_ref[...] = jnp.dot(xn, wa_ref[...], preferred_element_type=jnp.float32)
    bt_ref[...] = jnp.dot(xn, wb_ref[...], preferred_element_type=jnp.float32)
    hsn_ref[...] = jnp.dot(
        xn, w1_ref[...], preferred_element_type=jnp.float32) * dinv


_tc_combine3 = pl.pallas_call(
    _combine3_body,
    grid=(N // RB,),
    in_specs=[
        pl.BlockSpec((NC, RB, D), lambda i: (0, i, 0)),
        pl.BlockSpec((RB, D), lambda i: (i, 0)),
        pl.BlockSpec((RB, 1), lambda i: (i, 0)),
        pl.BlockSpec((1, D), lambda i: (0, 0)),
        pl.BlockSpec((D, D), lambda i: (0, 0)),
        pl.BlockSpec((D, D), lambda i: (0, 0)),
        pl.BlockSpec((D, D), lambda i: (0, 0)),
    ],
    out_specs=[
        pl.BlockSpec((RB, D), lambda i: (i, 0)),
        pl.BlockSpec((RB, D), lambda i: (i, 0)),
        pl.BlockSpec((RB, D), lambda i: (i, 0)),
    ],
    out_shape=[
        jax.ShapeDtypeStruct((N, D), jnp.float32),
        jax.ShapeDtypeStruct((N, D), jnp.float32),
        jax.ShapeDtypeStruct((N, D), jnp.float32),
    ],
)


def _edgec_body(attr_ref, wc_ref, be_ref, c_ref):
    i = pl.program_id(0)
    c = jnp.dot(attr_ref[...], wc_ref[...],
                preferred_element_type=jnp.float32) + be_ref[...]
    rows = i * EB + lax.broadcasted_iota(jnp.int32, (EB, D), 0)
    c_ref[...] = jnp.where(rows < E, c, NEG)


_tc_edgec = pl.pallas_call(
    _edgec_body,
    grid=(EP // EB,),
    in_specs=[
        pl.BlockSpec((EB, DE), lambda i: (i, 0)),
        pl.BlockSpec((DE, D), lambda i: (0, 0)),
        pl.BlockSpec((1, D), lambda i: (0, 0)),
    ],
    out_specs=pl.BlockSpec((EB, D), lambda i: (i, 0)),
    out_shape=jax.ShapeDtypeStruct((EP, D), jnp.float32),
)


def _head_body(p0_ref, p1_ref, wg_ref, bg_ref, out_ref):
    inv_e = 1.0 / E
    g0 = jnp.sum(p0_ref[...], axis=0, keepdims=True) * inv_e
    g1 = jnp.sum(p1_ref[...], axis=0, keepdims=True) * inv_e
    wg = wg_ref[...]
    bg = bg_ref[...]
    out_ref[0:1] = jnp.dot(g0, wg, preferred_element_type=jnp.float32) + bg
    out_ref[1:2] = jnp.dot(g1, wg, preferred_element_type=jnp.float32) + bg


_tc_head = pl.pallas_call(
    _head_body,
    out_shape=jax.ShapeDtypeStruct((T, 1), jnp.float32),
)


def kernel(x, edge_index, edge_attrs, W1, b1, W2, b2, W3, b3, We, be, Wg, bg):
    pad = EP - E
    src3 = jnp.concatenate(
        [edge_index[0], jnp.zeros((pad,), jnp.int32)]).reshape(NW, CH, K)
    dst3 = jnp.concatenate(
        [edge_index[1], jnp.full((pad,), N, jnp.int32)]).reshape(NW, CH, K)
    dst3e = jnp.concatenate(
        [edge_index[1], jnp.zeros((pad,), jnp.int32)]).reshape(NW, CH, K)
    attr_p = jnp.pad(edge_attrs, ((0, 0), (0, pad), (0, 0)))
    WeA, WeB, WeC = We[:D], We[D:2 * D], We[2 * D:]
    b1r, b2r, b3r = b1.reshape(1, D), b2.reshape(1, D), b3.reshape(1, D)

    deg_parts = _sc_deg(dst3).reshape(NW, NP)[:, :N].T
    dinv, hs = _tc_prep(deg_parts, x, W1)

    def conv_agg(hs_):
        out = _sc_conv(hs_, src3, dst3).reshape(NC, NS, ACR, D)
        return out[:, :, :RNG].reshape(NC, NP, D)

    psums = []
    for t in range(T):
        for bb, wn in ((b1r, W2), (b2r, W3)):
            hs = _tc_combine(conv_agg(hs), hs, dinv, bb, wn)
        a_t, b_t, hs = _tc_combine3(conv_agg(hs), hs, dinv, b3r, WeA, WeB, W1)
        c_t = _tc_edgec(attr_p[t], WeC, be.reshape(1, D))
        c3 = c_t.reshape(NW, CH, K, D)
        psums.append(_sc_edge(a_t, b_t, c3, src3, dst3e).reshape(NW, D))

    return _tc_head(psums[0], psums[1], Wg, bg.reshape(1, 1))


# speedup vs baseline: 1.0721x; 1.0721x over previous
import jax
import jax.numpy as jnp
from jax import lax
from jax.experimental import pallas as pl
from jax.experimental.pallas import tpu as pltpu
from jax.experimental.pallas import tpu_sc as plsc

N = 10000
E = 320000
D = 128
DE = 16
T = 2

NC = 2
NS = 16
NW = NC * NS
K = 128
CH = 79
EP = NW * CH * K
NP = 10240
RPS = NP // NS
NZ = RPS // K
NEG = -1.0e30

_MESH = dict(core_axis_name="c", subcore_axis_name="s", num_cores=NC,
             num_subcores=NS)


def _deg_body(dst_hbm, out_hbm, idx_v, deg_v):
    c = lax.axis_index("c")
    s = lax.axis_index("s")
    w = c * NS + s

    def fill_zero(i, _):
        deg_v[pl.ds(i * 16, 16)] = jnp.zeros((16,), jnp.float32)
        return 0
    lax.fori_loop(0, NP // 16, fill_zero, 0)

    pltpu.sync_copy(dst_hbm.at[w], idx_v)

    ones = jnp.ones((16,), jnp.float32)

    def chunk(j, _):
        for g in range(K // 16):
            idx = idx_v[j, pl.ds(g * 16, 16)]
            plsc.addupdate_scatter(deg_v, [idx], ones)
        return 0
    lax.fori_loop(0, CH, chunk, 0)

    pltpu.sync_copy(deg_v, out_hbm.at[c, s])


_sc_deg = pl.kernel(
    _deg_body,
    out_type=jax.ShapeDtypeStruct((NC, NS, NP), jnp.float32),
    mesh=plsc.VectorSubcoreMesh(**_MESH),
    compiler_params=pltpu.CompilerParams(needs_layout_passes=False),
    scratch_types=[
        pltpu.VMEM((CH, K), jnp.int32),
        pltpu.VMEM((NP,), jnp.float32),
    ],
)


RNG = NP // NS
ACR = RNG + 16
DB = 128
TMP = 272


def _conv_body(hs_hbm, src_hbm, dst_hbm, out_hbm, src_v, dst_v, buf_v,
               acc_v, tsrc_v, tdst_v, sem):
    c = lax.axis_index("c")
    s = lax.axis_index("s")
    lo = s * RNG

    def fill_zero(i, _):
        acc_v[pl.ds(i * 16, 16)] = jnp.zeros((16,), jnp.float32)
        return 0
    lax.fori_loop(0, ACR * D // 16, fill_zero, 0)

    lane = lax.iota(jnp.int32, 16)
    dummy_src = jnp.zeros((16,), jnp.int32)
    dummy_dst = jnp.full((16,), RNG * D, jnp.int32)
    for k in range(TMP // 16):
        tsrc_v[pl.ds(k * 16, 16)] = dummy_src
        tdst_v[pl.ds(k * 16, 16)] = dummy_dst

    def drain(b, _):
        pltpu.async_copy(
            hs_hbm.at[tsrc_v.at[pl.ds(b * DB, DB)]], buf_v, sem).wait()
        for lg in range(DB // 16):
            base = tdst_v[pl.ds(b * DB + lg * 16, 16)]
            rows = lane + (lg * 16)
            for col in range(D):
                v = plsc.load_gather(
                    buf_v, [rows, jnp.full((16,), col, jnp.int32)])
                plsc.addupdate_scatter(acc_v, [base + col], v)
        return 0

    def worker(wi, ptr):
        w2 = c * NS + wi
        pltpu.sync_copy(src_hbm.at[w2], src_v)
        pltpu.sync_copy(dst_hbm.at[w2], dst_v)

        def chunk(j, ptr):
            for g in range(K // 16):
                srcv = src_v[j, pl.ds(g * 16, 16)]
                dstv = dst_v[j, pl.ds(g * 16, 16)]
                m = (dstv >= lo) & (dstv < lo + RNG)
                mi = m.astype(jnp.int32)
                pos = plsc.cumsum(mi) - mi + ptr
                plsc.store_scatter(tsrc_v, [pos], srcv, mask=m)
                plsc.store_scatter(tdst_v, [pos], (dstv - lo) * D, mask=m)
                ptr = ptr + jnp.sum(mi)
            nb = ptr // DB
            lax.fori_loop(0, nb, drain, 0)
            rem = ptr - nb * DB

            @pl.when(nb > 0)
            def _shift():
                for k in range(DB // 16):
                    cs = tsrc_v[pl.ds(nb * DB + k * 16, 16)]
                    cd = tdst_v[pl.ds(nb * DB + k * 16, 16)]
                    tsrc_v[pl.ds(k * 16, 16)] = cs
                    tdst_v[pl.ds(k * 16, 16)] = cd
            return rem

        return lax.fori_loop(0, CH, chunk, ptr)

    ptr = lax.fori_loop(0, NS, worker, jnp.int32(0))
    for k in range(DB // 16):
        tsrc_v[pl.ds(ptr + k * 16, 16)] = dummy_src
        tdst_v[pl.ds(ptr + k * 16, 16)] = dummy_dst
    nb2 = (ptr + DB - 1) // DB
    lax.fori_loop(0, nb2, drain, 0)
    pltpu.sync_copy(acc_v, out_hbm.at[c, s])


_sc_conv = pl.kernel(
    _conv_body,
    out_type=jax.ShapeDtypeStruct((NC, NS, ACR * D), jnp.float32),
    mesh=plsc.VectorSubcoreMesh(**_MESH),
    compiler_params=pltpu.CompilerParams(needs_layout_passes=False),
    scratch_types=[
        pltpu.VMEM((CH, K), jnp.int32),
        pltpu.VMEM((CH, K), jnp.int32),
        pltpu.VMEM((DB, D), jnp.float32),
        pltpu.VMEM((ACR * D,), jnp.float32),
        pltpu.VMEM((TMP,), jnp.int32),
        pltpu.VMEM((TMP,), jnp.int32),
        pltpu.SemaphoreType.DMA,
    ],
)


def _edge_body(a_hbm, b_hbm, c_hbm, src_hbm, dst_hbm, out_hbm, src_v, dst_v,
               buf_a, buf_b, buf_c, out_v, sem_a, sem_b, sem_c):
    cc = lax.axis_index("c")
    s = lax.axis_index("s")
    w = cc * NS + s

    pltpu.sync_copy(src_hbm.at[w], src_v)
    pltpu.sync_copy(dst_hbm.at[w], dst_v)

    for g in range(8):
        out_v[0, pl.ds(g * 16, 16)] = jnp.zeros((16,), jnp.float32)

    def chunk(j, _):
        da = pltpu.async_copy(a_hbm.at[src_v.at[j]], buf_a, sem_a)
        db = pltpu.async_copy(b_hbm.at[dst_v.at[j]], buf_b, sem_b)
        dc = pltpu.async_copy(c_hbm.at[w, j], buf_c, sem_c)
        da.wait()
        db.wait()
        dc.wait()

        def edge(e, _):
            for g in range(8):
                av = buf_a[e, pl.ds(g * 16, 16)]
                bv = buf_b[e, pl.ds(g * 16, 16)]
                cv = buf_c[e, pl.ds(g * 16, 16)]
                out_v[0, pl.ds(g * 16, 16)] += jnp.maximum(av + bv + cv, 0.0)
            return 0
        return lax.fori_loop(0, K, edge, 0)

    lax.fori_loop(0, CH, chunk, 0)
    pltpu.sync_copy(out_v, out_hbm.at[w])


_sc_edge = pl.kernel(
    _edge_body,
    out_type=jax.ShapeDtypeStruct((NW, 1, D), jnp.float32),
    mesh=plsc.VectorSubcoreMesh(**_MESH),
    scratch_types=[
        pltpu.VMEM((CH, K), jnp.int32),
        pltpu.VMEM((CH, K), jnp.int32),
        pltpu.VMEM((K, D), jnp.float32),
        pltpu.VMEM((K, D), jnp.float32),
        pltpu.VMEM((K, D), jnp.float32),
        pltpu.VMEM((1, D), jnp.float32),
        pltpu.SemaphoreType.DMA,
        pltpu.SemaphoreType.DMA,
        pltpu.SemaphoreType.DMA,
    ],
)


RB = 1000
EB = 2048


def _prep_body(dp_ref, x_ref, w_ref, dinv_ref, hs_ref):
    deg = jnp.sum(dp_ref[...], axis=1, keepdims=True) + 1.0
    dinv = lax.rsqrt(deg)
    dinv_ref[...] = dinv
    h = jnp.dot(x_ref[...], w_ref[...], preferred_element_type=jnp.float32)
    hs_ref[...] = h * dinv


_tc_prep = pl.pallas_call(
    _prep_body,
    grid=(N // RB,),
    in_specs=[
        pl.BlockSpec((RB, NW), lambda i: (i, 0)),
        pl.BlockSpec((RB, D), lambda i: (i, 0)),
        pl.BlockSpec((D, D), lambda i: (0, 0)),
    ],
    out_specs=[
        pl.BlockSpec((RB, 1), lambda i: (i, 0)),
        pl.BlockSpec((RB, D), lambda i: (i, 0)),
    ],
    out_shape=[
        jax.ShapeDtypeStruct((N, 1), jnp.float32),
        jax.ShapeDtypeStruct((N, D), jnp.float32),
    ],
)


def _combine_body(agg_ref, hs_ref, dinv_ref, b_ref, wn_ref, out_ref):
    dinv = dinv_ref[...]
    xn = dinv * (agg_ref[0] + agg_ref[1] + hs_ref[...]) + b_ref[...]
    xn = jnp.maximum(xn, 0.0)
    out_ref[...] = jnp.dot(
        xn, wn_ref[...], preferred_element_type=jnp.float32) * dinv


_tc_combine = pl.pallas_call(
    _combine_body,
    grid=(N // RB,),
    in_specs=[
        pl.BlockSpec((NC, RB, D), lambda i: (0, i, 0)),
        pl.BlockSpec((RB, D), lambda i: (i, 0)),
        pl.BlockSpec((RB, 1), lambda i: (i, 0)),
        pl.BlockSpec((1, D), lambda i: (0, 0)),
        pl.BlockSpec((D, D), lambda i: (0, 0)),
    ],
    out_specs=pl.BlockSpec((RB, D), lambda i: (i, 0)),
    out_shape=jax.ShapeDtypeStruct((N, D), jnp.float32),
)


def _combine3_body(agg_ref, hs_ref, dinv_ref, b_ref, wa_ref, wb_ref, w1_ref,
                   a_ref, bt_ref, hsn_ref):
    dinv = dinv_ref[...]
    xn = dinv * (agg_ref[0] + agg_ref[1] + hs_ref[...]) + b_ref[...]
    xn = jnp.maximum(xn, 0.0)
    a_ref[...] = jnp.dot(xn, wa_ref[...], preferred_element_type=jnp.float32)
    bt_ref[...] = jnp.dot(xn, wb_ref[...], preferred_element_type=jnp.float32)
    hsn_ref[...] = jnp.dot(
        xn, w1_ref[...], preferred_element_type=jnp.float32) * dinv


_tc_combine3 = pl.pallas_call(
    _combine3_body,
    grid=(N // RB,),
    in_specs=[
        pl.BlockSpec((NC, RB, D), lambda i: (0, i, 0)),
        pl.BlockSpec((RB, D), lambda i: (i, 0)),
        pl.BlockSpec((RB, 1), lambda i: (i, 0)),
        pl.BlockSpec((1, D), lambda i: (0, 0)),
        pl.BlockSpec((D, D), lambda i: (0, 0)),
        pl.BlockSpec((D, D), lambda i: (0, 0)),
        pl.BlockSpec((D, D), lambda i: (0, 0)),
    ],
    out_specs=[
        pl.BlockSpec((RB, D), lambda i: (i, 0)),
        pl.BlockSpec((RB, D), lambda i: (i, 0)),
        pl.BlockSpec((RB, D), lambda i: (i, 0)),
    ],
    out_shape=[
        jax.ShapeDtypeStruct((N, D), jnp.float32),
        jax.ShapeDtypeStruct((N, D), jnp.float32),
        jax.ShapeDtypeStruct((N, D), jnp.float32),
    ],
)


def _edgec_body(attr_ref, wc_ref, be_ref, c_ref):
    i = pl.program_id(0)
    c = jnp.dot(attr_ref[...], wc_ref[...],
                preferred_element_type=jnp.float32) + be_ref[...]
    rows = i * EB + lax.broadcasted_iota(jnp.int32, (EB, D), 0)
    c_ref[...] = jnp.where(rows < E, c, NEG)


_tc_edgec = pl.pallas_call(
    _edgec_body,
    grid=(EP // EB,),
    in_specs=[
        pl.BlockSpec((EB, DE), lambda i: (i, 0)),
        pl.BlockSpec((DE, D), lambda i: (0, 0)),
        pl.BlockSpec((1, D), lambda i: (0, 0)),
    ],
    out_specs=pl.BlockSpec((EB, D), lambda i: (i, 0)),
    out_shape=jax.ShapeDtypeStruct((EP, D), jnp.float32),
)


def _head_body(p0_ref, p1_ref, wg_ref, bg_ref, out_ref):
    inv_e = 1.0 / E
    g0 = jnp.sum(p0_ref[...], axis=0, keepdims=True) * inv_e
    g1 = jnp.sum(p1_ref[...], axis=0, keepdims=True) * inv_e
    wg = wg_ref[...]
    bg = bg_ref[...]
    out_ref[0:1] = jnp.dot(g0, wg, preferred_element_type=jnp.float32) + bg
    out_ref[1:2] = jnp.dot(g1, wg, preferred_element_type=jnp.float32) + bg


_tc_head = pl.pallas_call(
    _head_body,
    out_shape=jax.ShapeDtypeStruct((T, 1), jnp.float32),
)


def kernel(x, edge_index, edge_attrs, W1, b1, W2, b2, W3, b3, We, be, Wg, bg):
    pad = EP - E
    src3 = jnp.concatenate(
        [edge_index[0], jnp.zeros((pad,), jnp.int32)]).reshape(NW, CH, K)
    dst3 = jnp.concatenate(
        [edge_index[1], jnp.full((pad,), N, jnp.int32)]).reshape(NW, CH, K)
    dst3e = jnp.concatenate(
        [edge_index[1], jnp.zeros((pad,), jnp.int32)]).reshape(NW, CH, K)
    attr_p = jnp.pad(edge_attrs, ((0, 0), (0, pad), (0, 0)))
    WeA, WeB, WeC = We[:D], We[D:2 * D], We[2 * D:]
    b1r, b2r, b3r = b1.reshape(1, D), b2.reshape(1, D), b3.reshape(1, D)

    deg_parts = _sc_deg(dst3).reshape(NW, NP)[:, :N].T
    dinv, hs = _tc_prep(deg_parts, x, W1)

    def conv_agg(hs_):
        out = _sc_conv(hs_, src3, dst3).reshape(NC, NS, ACR, D)
        return out[:, :, :RNG].reshape(NC, NP, D)

    psums = []
    for t in range(T):
        for bb, wn in ((b1r, W2), (b2r, W3)):
            hs = _tc_combine(conv_agg(hs), hs, dinv, bb, wn)
        a_t, b_t, hs = _tc_combine3(conv_agg(hs), hs, dinv, b3r, WeA, WeB, W1)
        c_t = _tc_edgec(attr_p[t], WeC, be.reshape(1, D))
        c3 = c_t.reshape(NW, CH, K, D)
        psums.append(_sc_edge(a_t, b_t, c3, src3, dst3e).reshape(NW, D))

    return _tc_head(psums[0], psums[1], Wg, bg.reshape(1, 1))
